# baseline (speedup 1.0000x reference)
import jax
import jax.numpy as jnp
from jax import lax
from jax.experimental import pallas as pl
from jax.experimental.pallas import tpu as pltpu
from jax.experimental.pallas import tpu_sc as plsc

N = 10000
E = 320000
IN_DIM = 128
OUT_DIM = 128
EDGE_DIM = 16
LATENT = 96
N_AGGR = 8
EPS = 1e-7

NUM_CORES = 2
NUM_SUBCORES = 16
NUM_TILES = NUM_CORES * NUM_SUBCORES
CHUNK = 80
E_PAD = 327680
E_PER_TILE = E_PAD // NUM_TILES
NCHUNKS = E_PER_TILE // CHUNK
AGG_ROWS = 10240
ROWS_PER_SUB = AGG_ROWS // NUM_SUBCORES

_SELU_ALPHA = 1.6732632423543772848170429916717
_SELU_SCALE = 1.0507009873554804934193349852946


def _selu(x):
  return _SELU_SCALE * jnp.where(
      x > 0, x, _SELU_ALPHA * (jnp.exp(jnp.minimum(x, 0.0)) - 1.0))


def _matmul_body(x_ref, w_ref, o_ref):
  o_ref[...] = jnp.dot(x_ref[...], w_ref[...],
                       preferred_element_type=jnp.float32)


def _matmul(x, w):
  m, _ = x.shape
  _, n = w.shape
  return pl.pallas_call(
      _matmul_body,
      out_shape=jax.ShapeDtypeStruct((m, n), jnp.float32),
  )(x, w)


def _edge_embed(edge_attr, w_edge):
  blk = 8192
  grid = E_PAD // blk
  return pl.pallas_call(
      _matmul_body,
      grid=(grid,),
      in_specs=[
          pl.BlockSpec((blk, EDGE_DIM), lambda i: (i, 0)),
          pl.BlockSpec((EDGE_DIM, LATENT), lambda i: (0, 0)),
      ],
      out_specs=pl.BlockSpec((blk, LATENT), lambda i: (i, 0)),
      out_shape=jax.ShapeDtypeStruct((E_PAD, LATENT), jnp.float32),
  )(edge_attr, w_edge)


def _mlp_body(z_ref, aggr_ref, w1_ref, w2_ref, o_ref):
  z = z_ref[...]
  out = z + aggr_ref[0:N, :] + aggr_ref[AGG_ROWS:AGG_ROWS + N, :]
  h = jnp.maximum(
      jnp.dot(out, w1_ref[...], preferred_element_type=jnp.float32), 0.0)
  o_ref[...] = _selu(
      jnp.dot(h, w2_ref[...], preferred_element_type=jnp.float32))


def _mlp(z, aggr, w1, w2):
  return pl.pallas_call(
      _mlp_body,
      out_shape=jax.ShapeDtypeStruct((N, LATENT), jnp.float32),
  )(z, aggr, w1, w2)


Z_ROWS_PER_SUB = N // NUM_SUBCORES


def _sc_edge_body(z_hbm, src_hbm, dst_hbm, ea_hbm, out_hbm,
                  src0, dst0, ea0, rows0, src1, dst1, ea1, rows1,
                  aggr_sh, lsem0, lsem1, gsem0, gsem1):
  cid = lax.axis_index("c")
  sid = lax.axis_index("s")
  wid = sid * NUM_CORES + cid

  srcs = (src0, src1)
  dsts = (dst0, dst1)
  eas = (ea0, ea1)
  rowss = (rows0, rows1)
  lsems = (lsem0, lsem1)
  gsems = (gsem0, gsem1)

  zeros16 = jnp.zeros((16,), jnp.float32)

  @plsc.parallel_loop(0, CHUNK, step=1)
  def _zero_row(i):
    for j in range(LATENT // 16):
      rows0[i, pl.ds(j * 16, 16)] = zeros16

  for k in range(ROWS_PER_SUB // CHUNK):
    pltpu.sync_copy(rows0,
                    aggr_sh.at[pl.ds(sid * ROWS_PER_SUB + k * CHUNK, CHUNK)])
  plsc.subcore_barrier()

  base0 = wid * E_PER_TILE

  def _start_loads(c, b):
    base = base0 + c * CHUNK
    pltpu.async_copy(src_hbm.at[pl.ds(base, CHUNK)], srcs[b], lsems[b])
    pltpu.async_copy(dst_hbm.at[pl.ds(base, CHUNK)], dsts[b], lsems[b])
    pltpu.async_copy(ea_hbm.at[pl.ds(base, CHUNK)], eas[b], lsems[b])

  def _wait_loads(c, b):
    base = base0 + c * CHUNK
    pltpu.make_async_copy(src_hbm.at[pl.ds(base, CHUNK)], srcs[b],
                          lsems[b]).wait()
    pltpu.make_async_copy(dst_hbm.at[pl.ds(base, CHUNK)], dsts[b],
                          lsems[b]).wait()
    pltpu.make_async_copy(ea_hbm.at[pl.ds(base, CHUNK)], eas[b],
                          lsems[b]).wait()

  def _start_gather(b):
    pltpu.async_copy(z_hbm.at[srcs[b]], rowss[b], gsems[b])

  def _wait_gather(b):
    pltpu.make_async_copy(z_hbm.at[srcs[b]], rowss[b], gsems[b]).wait()

  def _compute(b):
    rows = rowss[b]
    ea = eas[b]

    @plsc.parallel_loop(0, CHUNK, step=1, unroll=2)
    def _row(i):
      for j in range(LATENT // 16):
        sl = pl.ds(j * 16, 16)
        rows[i, sl] = jnp.maximum(rows[i, sl] + ea[i, sl], 0.0) + EPS

  def _scatter(b):
    pltpu.sync_copy(rowss[b], aggr_sh.at[dsts[b]], add=True)

  _start_loads(0, 0)

  def _pair_steps(e, prefetch):
    _wait_loads(e, 0)
    _start_gather(0)
    _start_loads(e + 1, 1)
    _wait_gather(0)
    _compute(0)
    _wait_loads(e + 1, 1)
    _start_gather(1)
    _scatter(0)
    if prefetch:
      _start_loads(e + 2, 0)
    _wait_gather(1)
    _compute(1)
    _scatter(1)

  def _pair(i, _):
    _pair_steps(2 * i, True)
    return 0

  lax.fori_loop(0, NCHUNKS // 2 - 1, _pair, 0)
  _pair_steps(NCHUNKS - 2, False)

  plsc.subcore_barrier()

  row0 = sid * ROWS_PER_SUB
  pltpu.sync_copy(aggr_sh.at[pl.ds(row0, ROWS_PER_SUB)],
                  out_hbm.at[pl.ds(cid * AGG_ROWS + row0, ROWS_PER_SUB)])


def _sc_edge_pass(z, src, dst, ea):
  mesh = plsc.VectorSubcoreMesh(
      core_axis_name="c", subcore_axis_name="s",
      num_cores=NUM_CORES, num_subcores=NUM_SUBCORES)
  call = pl.kernel(
      _sc_edge_body,
      out_type=jax.ShapeDtypeStruct((NUM_CORES * AGG_ROWS, LATENT),
                                    jnp.float32),
      mesh=mesh,
      scratch_types=[
          pltpu.VMEM((CHUNK,), jnp.int32),
          pltpu.VMEM((CHUNK,), jnp.int32),
          pltpu.VMEM((CHUNK, LATENT), jnp.float32),
          pltpu.VMEM((CHUNK, LATENT), jnp.float32),
          pltpu.VMEM((CHUNK,), jnp.int32),
          pltpu.VMEM((CHUNK,), jnp.int32),
          pltpu.VMEM((CHUNK, LATENT), jnp.float32),
          pltpu.VMEM((CHUNK, LATENT), jnp.float32),
          pltpu.VMEM_SHARED((AGG_ROWS, LATENT), jnp.float32),
          pltpu.SemaphoreType.DMA,
          pltpu.SemaphoreType.DMA,
          pltpu.SemaphoreType.DMA,
          pltpu.SemaphoreType.DMA,
      ],
      compiler_params=pltpu.CompilerParams(
          use_tc_tiling_on_sc=False,
          internal_scratch_in_bytes=64 * 1024),
  )
  return call(z, src, dst, ea)


def kernel(x, y, edge_index, edge_attr, W_node_in, W_edge, gcn_W1, gcn_W2,
           W_node_out):
  pad = E_PAD - E
  src = jnp.concatenate([edge_index[0], jnp.zeros((pad,), jnp.int32)])
  dst = jnp.concatenate(
      [edge_index[1],
       N + (jnp.arange(pad, dtype=jnp.int32) % (AGG_ROWS - N))])
  edge_attr_p = jnp.concatenate(
      [edge_attr, jnp.zeros((pad, EDGE_DIM), jnp.float32)])
  ea = _edge_embed(edge_attr_p, W_edge)
  z = _matmul(x, W_node_in)
  for i in range(N_AGGR):
    aggr = _sc_edge_pass(z, src, dst, ea)
    z = _mlp(z, aggr, gcn_W1[i], gcn_W2[i])
  y_predict = _matmul(z, W_node_out)
  return (y, y_predict)


# speedup vs baseline: 1.8891x; 1.8891x over previous
import jax
import jax.numpy as jnp
from jax import lax
from jax.experimental import pallas as pl
from jax.experimental.pallas import tpu as pltpu
from jax.experimental.pallas import tpu_sc as plsc

N = 10000
E = 320000
IN_DIM = 128
OUT_DIM = 128
EDGE_DIM = 16
LATENT = 96
N_AGGR = 8
EPS = 1e-7

NUM_CORES = 2
NUM_SUBCORES = 16
NUM_TILES = NUM_CORES * NUM_SUBCORES
CHUNK = 80
E_PAD = E
E_PER_TILE = E_PAD // NUM_TILES
NCHUNKS = E_PER_TILE // CHUNK
AGG_ROWS = 10240
ROWS_PER_SUB = AGG_ROWS // NUM_SUBCORES

_SELU_ALPHA = 1.6732632423543772848170429916717
_SELU_SCALE = 1.0507009873554804934193349852946


def _selu(x):
  return _SELU_SCALE * jnp.where(
      x > 0, x, _SELU_ALPHA * (jnp.exp(jnp.minimum(x, 0.0)) - 1.0))


def _matmul_body(x_ref, w_ref, o_ref):
  o_ref[...] = jnp.dot(x_ref[...], w_ref[...],
                       preferred_element_type=jnp.float32)


def _matmul(x, w):
  m, _ = x.shape
  _, n = w.shape
  return pl.pallas_call(
      _matmul_body,
      out_shape=jax.ShapeDtypeStruct((m, n), jnp.float32),
  )(x, w)


def _edge_embed(edge_attr, w_edge):
  blk = 8000
  grid = E_PAD // blk
  return pl.pallas_call(
      _matmul_body,
      grid=(grid,),
      in_specs=[
          pl.BlockSpec((blk, EDGE_DIM), lambda i: (i, 0)),
          pl.BlockSpec((EDGE_DIM, LATENT), lambda i: (0, 0)),
      ],
      out_specs=pl.BlockSpec((blk, LATENT), lambda i: (i, 0)),
      out_shape=jax.ShapeDtypeStruct((E_PAD, LATENT), jnp.float32),
  )(edge_attr, w_edge)


def _mlp_body(z_ref, aggr_ref, w1_ref, w2_ref, o_ref):
  z = z_ref[...]
  out = z + aggr_ref[0:N, :] + aggr_ref[AGG_ROWS:AGG_ROWS + N, :]
  h = jnp.maximum(
      jnp.dot(out, w1_ref[...], preferred_element_type=jnp.float32), 0.0)
  o_ref[...] = _selu(
      jnp.dot(h, w2_ref[...], preferred_element_type=jnp.float32))


def _mlp(z, aggr, w1, w2):
  return pl.pallas_call(
      _mlp_body,
      out_shape=jax.ShapeDtypeStruct((N, LATENT), jnp.float32),
  )(z, aggr, w1, w2)


Z_ROWS_PER_SUB = N // NUM_SUBCORES


def _sc_edge_body(z_hbm, src_hbm, dst_hbm, ea_hbm, out_hbm,
                  src0, dst0, ea0, rows0, src1, dst1, ea1, rows1,
                  aggr_sh, lsem0, lsem1, gsem0, gsem1):
  cid = lax.axis_index("c")
  sid = lax.axis_index("s")
  wid = sid * NUM_CORES + cid

  srcs = (src0, src1)
  dsts = (dst0, dst1)
  eas = (ea0, ea1)
  rowss = (rows0, rows1)
  lsems = (lsem0, lsem1)
  gsems = (gsem0, gsem1)

  zeros16 = jnp.zeros((16,), jnp.float32)

  @plsc.parallel_loop(0, CHUNK, step=1)
  def _zero_row(i):
    for j in range(LATENT // 16):
      rows0[i, pl.ds(j * 16, 16)] = zeros16

  for k in range(ROWS_PER_SUB // CHUNK):
    pltpu.sync_copy(rows0,
                    aggr_sh.at[pl.ds(sid * ROWS_PER_SUB + k * CHUNK, CHUNK)])
  plsc.subcore_barrier()

  base0 = wid * E_PER_TILE

  def _start_loads(c, b):
    base = base0 + c * CHUNK
    pltpu.async_copy(src_hbm.at[pl.ds(base, CHUNK)], srcs[b], lsems[b])
    pltpu.async_copy(dst_hbm.at[pl.ds(base, CHUNK)], dsts[b], lsems[b])
    pltpu.async_copy(ea_hbm.at[pl.ds(base, CHUNK)], eas[b], lsems[b])

  def _wait_loads(c, b):
    base = base0 + c * CHUNK
    pltpu.make_async_copy(src_hbm.at[pl.ds(base, CHUNK)], srcs[b],
                          lsems[b]).wait()
    pltpu.make_async_copy(dst_hbm.at[pl.ds(base, CHUNK)], dsts[b],
                          lsems[b]).wait()
    pltpu.make_async_copy(ea_hbm.at[pl.ds(base, CHUNK)], eas[b],
                          lsems[b]).wait()

  def _start_gather(b):
    pltpu.async_copy(z_hbm.at[srcs[b]], rowss[b], gsems[b])

  def _wait_gather(b):
    pltpu.make_async_copy(z_hbm.at[srcs[b]], rowss[b], gsems[b]).wait()

  def _compute(b):
    rows = rowss[b]
    ea = eas[b]

    @plsc.parallel_loop(0, CHUNK, step=1, unroll=2)
    def _row(i):
      for j in range(LATENT // 16):
        sl = pl.ds(j * 16, 16)
        rows[i, sl] = jnp.maximum(rows[i, sl] + ea[i, sl], 0.0) + EPS

  def _scatter(b):
    pltpu.sync_copy(rowss[b], aggr_sh.at[dsts[b]], add=True)

  _start_loads(0, 0)

  def _pair_steps(e, prefetch):
    _wait_loads(e, 0)
    _start_gather(0)
    _start_loads(e + 1, 1)
    _wait_gather(0)
    _compute(0)
    _wait_loads(e + 1, 1)
    _start_gather(1)
    _scatter(0)
    if prefetch:
      _start_loads(e + 2, 0)
    _wait_gather(1)
    _compute(1)
    _scatter(1)

  def _pair(i, _):
    _pair_steps(2 * i, True)
    return 0

  if NCHUNKS % 2 == 0:
    lax.fori_loop(0, NCHUNKS // 2 - 1, _pair, 0)
    _pair_steps(NCHUNKS - 2, False)
  else:
    lax.fori_loop(0, (NCHUNKS - 1) // 2, _pair, 0)
    _wait_loads(NCHUNKS - 1, 0)
    _start_gather(0)
    _wait_gather(0)
    _compute(0)
    _scatter(0)

  plsc.subcore_barrier()

  row0 = sid * ROWS_PER_SUB
  pltpu.sync_copy(aggr_sh.at[pl.ds(row0, ROWS_PER_SUB)],
                  out_hbm.at[pl.ds(cid * AGG_ROWS + row0, ROWS_PER_SUB)])


def _sc_edge_pass(z, src, dst, ea):
  mesh = plsc.VectorSubcoreMesh(
      core_axis_name="c", subcore_axis_name="s",
      num_cores=NUM_CORES, num_subcores=NUM_SUBCORES)
  call = pl.kernel(
      _sc_edge_body,
      out_type=jax.ShapeDtypeStruct((NUM_CORES * AGG_ROWS, LATENT),
                                    jnp.float32),
      mesh=mesh,
      scratch_types=[
          pltpu.VMEM((CHUNK,), jnp.int32),
          pltpu.VMEM((CHUNK,), jnp.int32),
          pltpu.VMEM((CHUNK, LATENT), jnp.float32),
          pltpu.VMEM((CHUNK, LATENT), jnp.float32),
          pltpu.VMEM((CHUNK,), jnp.int32),
          pltpu.VMEM((CHUNK,), jnp.int32),
          pltpu.VMEM((CHUNK, LATENT), jnp.float32),
          pltpu.VMEM((CHUNK, LATENT), jnp.float32),
          pltpu.VMEM_SHARED((AGG_ROWS, LATENT), jnp.float32),
          pltpu.SemaphoreType.DMA,
          pltpu.SemaphoreType.DMA,
          pltpu.SemaphoreType.DMA,
          pltpu.SemaphoreType.DMA,
      ],
      compiler_params=pltpu.CompilerParams(
          use_tc_tiling_on_sc=False,
          internal_scratch_in_bytes=64 * 1024),
  )
  return call(z, src, dst, ea)


def kernel(x, y, edge_index, edge_attr, W_node_in, W_edge, gcn_W1, gcn_W2,
           W_node_out):
  src = edge_index[0]
  dst = edge_index[1]
  ea = _edge_embed(edge_attr, W_edge)
  z = _matmul(x, W_node_in)
  for i in range(N_AGGR):
    aggr = _sc_edge_pass(z, src, dst, ea)
    z = _mlp(z, aggr, gcn_W1[i], gcn_W2[i])
  y_predict = _matmul(z, W_node_out)
  return (y, y_predict)


# speedup vs baseline: 1.8932x; 1.0022x over previous
import jax
import jax.numpy as jnp
from jax import lax
from jax.experimental import pallas as pl
from jax.experimental.pallas import tpu as pltpu
from jax.experimental.pallas import tpu_sc as plsc

N = 10000
E = 320000
IN_DIM = 128
OUT_DIM = 128
EDGE_DIM = 16
LATENT = 96
N_AGGR = 8
EPS = 1e-7

NUM_CORES = 2
NUM_SUBCORES = 16
NUM_TILES = NUM_CORES * NUM_SUBCORES
CHUNK = 80
E_PAD = E
E_PER_TILE = E_PAD // NUM_TILES
NCHUNKS = E_PER_TILE // CHUNK
AGG_ROWS = 10240
ROWS_PER_SUB = AGG_ROWS // NUM_SUBCORES

_SELU_ALPHA = 1.6732632423543772848170429916717
_SELU_SCALE = 1.0507009873554804934193349852946


def _selu(x):
  return _SELU_SCALE * jnp.where(
      x > 0, x, _SELU_ALPHA * (jnp.exp(jnp.minimum(x, 0.0)) - 1.0))


def _matmul_body(x_ref, w_ref, o_ref):
  o_ref[...] = jnp.dot(x_ref[...], w_ref[...],
                       preferred_element_type=jnp.float32)


def _matmul(x, w):
  m, _ = x.shape
  _, n = w.shape
  return pl.pallas_call(
      _matmul_body,
      out_shape=jax.ShapeDtypeStruct((m, n), jnp.float32),
  )(x, w)


def _edge_embed(edge_attr, w_edge):
  blk = 8000
  grid = E_PAD // blk
  return pl.pallas_call(
      _matmul_body,
      grid=(grid,),
      in_specs=[
          pl.BlockSpec((blk, EDGE_DIM), lambda i: (i, 0)),
          pl.BlockSpec((EDGE_DIM, LATENT), lambda i: (0, 0)),
      ],
      out_specs=pl.BlockSpec((blk, LATENT), lambda i: (i, 0)),
      out_shape=jax.ShapeDtypeStruct((E_PAD, LATENT), jnp.float32),
  )(edge_attr, w_edge)


def _mlp_body(z_ref, aggr_ref, w1_ref, w2_ref, o_ref):
  z = z_ref[...]
  out = z + aggr_ref[0:N, :] + aggr_ref[AGG_ROWS:AGG_ROWS + N, :]
  h = jnp.maximum(
      jnp.dot(out, w1_ref[...], preferred_element_type=jnp.float32), 0.0)
  o_ref[...] = _selu(
      jnp.dot(h, w2_ref[...], preferred_element_type=jnp.float32))


def _mlp(z, aggr, w1, w2):
  return pl.pallas_call(
      _mlp_body,
      out_shape=jax.ShapeDtypeStruct((N, LATENT), jnp.float32),
  )(z, aggr, w1, w2)


Z_ROWS_PER_SUB = N // NUM_SUBCORES


def _sc_edge_body(z_hbm, src_hbm, dst_hbm, ea_hbm, out_hbm,
                  src0, dst0, ea0, rows0, src1, dst1, ea1, rows1,
                  aggr_sh, lsem0, lsem1, gsem0, gsem1):
  cid = lax.axis_index("c")
  sid = lax.axis_index("s")
  wid = sid * NUM_CORES + cid

  srcs = (src0, src1)
  dsts = (dst0, dst1)
  eas = (ea0, ea1)
  rowss = (rows0, rows1)
  lsems = (lsem0, lsem1)
  gsems = (gsem0, gsem1)

  zeros16 = jnp.zeros((16,), jnp.float32)

  @plsc.parallel_loop(0, CHUNK, step=1)
  def _zero_row(i):
    for j in range(LATENT // 16):
      rows0[i, pl.ds(j * 16, 16)] = zeros16

  for k in range(ROWS_PER_SUB // CHUNK):
    pltpu.sync_copy(rows0,
                    aggr_sh.at[pl.ds(sid * ROWS_PER_SUB + k * CHUNK, CHUNK)])
  plsc.subcore_barrier()

  base0 = wid * E_PER_TILE

  def _start_loads(c, b):
    base = base0 + c * CHUNK
    pltpu.async_copy(src_hbm.at[pl.ds(base, CHUNK)], srcs[b], lsems[b])
    pltpu.async_copy(dst_hbm.at[pl.ds(base, CHUNK)], dsts[b], lsems[b])
    pltpu.async_copy(ea_hbm.at[pl.ds(base, CHUNK)], eas[b], lsems[b])

  def _wait_loads(c, b):
    base = base0 + c * CHUNK
    pltpu.make_async_copy(src_hbm.at[pl.ds(base, CHUNK)], srcs[b],
                          lsems[b]).wait()
    pltpu.make_async_copy(dst_hbm.at[pl.ds(base, CHUNK)], dsts[b],
                          lsems[b]).wait()
    pltpu.make_async_copy(ea_hbm.at[pl.ds(base, CHUNK)], eas[b],
                          lsems[b]).wait()

  def _start_gather(b):
    pltpu.async_copy(z_hbm.at[srcs[b]], rowss[b], gsems[b])

  def _wait_gather(b):
    pltpu.make_async_copy(z_hbm.at[srcs[b]], rowss[b], gsems[b]).wait()

  def _compute(b):
    rows = rowss[b]
    ea = eas[b]

    @plsc.parallel_loop(0, CHUNK, step=1, unroll=4)
    def _row(i):
      for j in range(LATENT // 16):
        sl = pl.ds(j * 16, 16)
        rows[i, sl] = jnp.maximum(rows[i, sl] + ea[i, sl], 0.0) + EPS

  def _scatter(b):
    pltpu.sync_copy(rowss[b], aggr_sh.at[dsts[b]], add=True)

  _start_loads(0, 0)

  def _pair_steps(e, prefetch):
    _wait_loads(e, 0)
    _start_gather(0)
    _start_loads(e + 1, 1)
    _wait_gather(0)
    _compute(0)
    _wait_loads(e + 1, 1)
    _start_gather(1)
    _scatter(0)
    if prefetch:
      _start_loads(e + 2, 0)
    _wait_gather(1)
    _compute(1)
    _scatter(1)

  def _pair(i, _):
    _pair_steps(2 * i, True)
    return 0

  if NCHUNKS % 2 == 0:
    lax.fori_loop(0, NCHUNKS // 2 - 1, _pair, 0)
    _pair_steps(NCHUNKS - 2, False)
  else:
    lax.fori_loop(0, (NCHUNKS - 1) // 2, _pair, 0)
    _wait_loads(NCHUNKS - 1, 0)
    _start_gather(0)
    _wait_gather(0)
    _compute(0)
    _scatter(0)

  plsc.subcore_barrier()

  row0 = sid * ROWS_PER_SUB
  pltpu.sync_copy(aggr_sh.at[pl.ds(row0, ROWS_PER_SUB)],
                  out_hbm.at[pl.ds(cid * AGG_ROWS + row0, ROWS_PER_SUB)])


def _sc_edge_pass(z, src, dst, ea):
  mesh = plsc.VectorSubcoreMesh(
      core_axis_name="c", subcore_axis_name="s",
      num_cores=NUM_CORES, num_subcores=NUM_SUBCORES)
  call = pl.kernel(
      _sc_edge_body,
      out_type=jax.ShapeDtypeStruct((NUM_CORES * AGG_ROWS, LATENT),
                                    jnp.float32),
      mesh=mesh,
      scratch_types=[
          pltpu.VMEM((CHUNK,), jnp.int32),
          pltpu.VMEM((CHUNK,), jnp.int32),
          pltpu.VMEM((CHUNK, LATENT), jnp.float32),
          pltpu.VMEM((CHUNK, LATENT), jnp.float32),
          pltpu.VMEM((CHUNK,), jnp.int32),
          pltpu.VMEM((CHUNK,), jnp.int32),
          pltpu.VMEM((CHUNK, LATENT), jnp.float32),
          pltpu.VMEM((CHUNK, LATENT), jnp.float32),
          pltpu.VMEM_SHARED((AGG_ROWS, LATENT), jnp.float32),
          pltpu.SemaphoreType.DMA,
          pltpu.SemaphoreType.DMA,
          pltpu.SemaphoreType.DMA,
          pltpu.SemaphoreType.DMA,
      ],
      compiler_params=pltpu.CompilerParams(
          use_tc_tiling_on_sc=False,
          internal_scratch_in_bytes=64 * 1024),
  )
  return call(z, src, dst, ea)


def kernel(x, y, edge_index, edge_attr, W_node_in, W_edge, gcn_W1, gcn_W2,
           W_node_out):
  src = edge_index[0]
  dst = edge_index[1]
  ea = _edge_embed(edge_attr, W_edge)
  z = _matmul(x, W_node_in)
  for i in range(N_AGGR):
    aggr = _sc_edge_pass(z, src, dst, ea)
    z = _mlp(z, aggr, gcn_W1[i], gcn_W2[i])
  y_predict = _matmul(z, W_node_out)
  return (y, y_predict)


# speedup vs baseline: 2.0256x; 1.0700x over previous
import jax
import jax.numpy as jnp
from jax import lax
from jax.experimental import pallas as pl
from jax.experimental.pallas import tpu as pltpu
from jax.experimental.pallas import tpu_sc as plsc

N = 10000
E = 320000
IN_DIM = 128
OUT_DIM = 128
EDGE_DIM = 16
LATENT = 96
N_AGGR = 8
EPS = 1e-7

NUM_CORES = 2
NUM_SUBCORES = 16
NUM_TILES = NUM_CORES * NUM_SUBCORES
CHUNK = 80
E_PAD = E
E_PER_TILE = E_PAD // NUM_TILES
NCHUNKS = E_PER_TILE // CHUNK
AGG_ROWS = 10240
ROWS_PER_SUB = AGG_ROWS // NUM_SUBCORES

_SELU_ALPHA = 1.6732632423543772848170429916717
_SELU_SCALE = 1.0507009873554804934193349852946


def _selu(x):
  return _SELU_SCALE * jnp.where(
      x > 0, x, _SELU_ALPHA * (jnp.exp(jnp.minimum(x, 0.0)) - 1.0))


def _matmul_body(x_ref, w_ref, o_ref):
  o_ref[...] = jnp.dot(x_ref[...], w_ref[...],
                       preferred_element_type=jnp.float32)


def _matmul(x, w):
  m, _ = x.shape
  _, n = w.shape
  return pl.pallas_call(
      _matmul_body,
      out_shape=jax.ShapeDtypeStruct((m, n), jnp.float32),
  )(x, w)


def _edge_embed_body(x_ref, w_ref, o_ref):
  y = jnp.dot(x_ref[...], w_ref[...], preferred_element_type=jnp.float32)
  o_ref[...] = y.astype(jnp.bfloat16)


_EA_PERM = [32 * g + 16 * p + k
            for g in range(LATENT // 32) for k in range(16) for p in range(2)]


def _edge_embed(edge_attr, w_edge):
  blk = 8000
  grid = E_PAD // blk
  return pl.pallas_call(
      _edge_embed_body,
      grid=(grid,),
      in_specs=[
          pl.BlockSpec((blk, EDGE_DIM), lambda i: (i, 0)),
          pl.BlockSpec((EDGE_DIM, LATENT), lambda i: (0, 0)),
      ],
      out_specs=pl.BlockSpec((blk, LATENT), lambda i: (i, 0)),
      out_shape=jax.ShapeDtypeStruct((E_PAD, LATENT), jnp.bfloat16),
  )(edge_attr, w_edge)


def _mlp_body(z_ref, aggr_ref, w1_ref, w2_ref, o_ref):
  z = z_ref[...]
  out = z + aggr_ref[0:N, :] + aggr_ref[AGG_ROWS:AGG_ROWS + N, :]
  h = jnp.maximum(
      jnp.dot(out, w1_ref[...], preferred_element_type=jnp.float32), 0.0)
  o_ref[...] = _selu(
      jnp.dot(h, w2_ref[...], preferred_element_type=jnp.float32))


def _mlp(z, aggr, w1, w2):
  return pl.pallas_call(
      _mlp_body,
      out_shape=jax.ShapeDtypeStruct((N, LATENT), jnp.float32),
  )(z, aggr, w1, w2)


Z_ROWS_PER_SUB = N // NUM_SUBCORES


def _sc_edge_body(z_hbm, src_hbm, dst_hbm, ea_hbm, out_hbm,
                  src0, dst0, ea0, rows0, src1, dst1, ea1, rows1,
                  aggr_sh, lsem0, lsem1, gsem0, gsem1):
  cid = lax.axis_index("c")
  sid = lax.axis_index("s")
  wid = sid * NUM_CORES + cid

  srcs = (src0, src1)
  dsts = (dst0, dst1)
  eas = (ea0, ea1)
  rowss = (rows0, rows1)
  lsems = (lsem0, lsem1)
  gsems = (gsem0, gsem1)

  zeros16 = jnp.zeros((16,), jnp.float32)

  @plsc.parallel_loop(0, CHUNK, step=1)
  def _zero_row(i):
    for j in range(LATENT // 16):
      rows0[i, pl.ds(j * 16, 16)] = zeros16

  for k in range(ROWS_PER_SUB // CHUNK):
    pltpu.sync_copy(rows0,
                    aggr_sh.at[pl.ds(sid * ROWS_PER_SUB + k * CHUNK, CHUNK)])
  plsc.subcore_barrier()

  base0 = wid * E_PER_TILE

  def _start_loads(c, b):
    base = base0 + c * CHUNK
    pltpu.async_copy(src_hbm.at[pl.ds(base, CHUNK)], srcs[b], lsems[b])
    pltpu.async_copy(dst_hbm.at[pl.ds(base, CHUNK)], dsts[b], lsems[b])
    pltpu.async_copy(ea_hbm.at[pl.ds(base, CHUNK)], eas[b], lsems[b])

  def _wait_loads(c, b):
    base = base0 + c * CHUNK
    pltpu.make_async_copy(src_hbm.at[pl.ds(base, CHUNK)], srcs[b],
                          lsems[b]).wait()
    pltpu.make_async_copy(dst_hbm.at[pl.ds(base, CHUNK)], dsts[b],
                          lsems[b]).wait()
    pltpu.make_async_copy(ea_hbm.at[pl.ds(base, CHUNK)], eas[b],
                          lsems[b]).wait()

  def _start_gather(b):
    pltpu.async_copy(z_hbm.at[srcs[b]], rowss[b], gsems[b])

  def _wait_gather(b):
    pltpu.make_async_copy(z_hbm.at[srcs[b]], rowss[b], gsems[b]).wait()

  def _compute(b):
    rows = rowss[b]
    ea = eas[b]

    @plsc.parallel_loop(0, CHUNK, step=1, unroll=4)
    def _row(i):
      for j in range(LATENT // 32):
        e0, e1 = plsc.unpack(ea[i, pl.ds(j * 32, 32)],
                             format=plsc.PackFormat.INTERLEAVED)
        sl0 = pl.ds(j * 32, 16)
        sl1 = pl.ds(j * 32 + 16, 16)
        rows[i, sl0] = jnp.maximum(rows[i, sl0] + e0, 0.0) + EPS
        rows[i, sl1] = jnp.maximum(rows[i, sl1] + e1, 0.0) + EPS

  def _scatter(b):
    pltpu.sync_copy(rowss[b], aggr_sh.at[dsts[b]], add=True)

  _start_loads(0, 0)

  def _pair_steps(e, prefetch):
    _wait_loads(e, 0)
    _start_gather(0)
    _start_loads(e + 1, 1)
    _wait_gather(0)
    _compute(0)
    _wait_loads(e + 1, 1)
    _start_gather(1)
    _scatter(0)
    if prefetch:
      _start_loads(e + 2, 0)
    _wait_gather(1)
    _compute(1)
    _scatter(1)

  def _pair(i, _):
    _pair_steps(2 * i, True)
    return 0

  if NCHUNKS % 2 == 0:
    lax.fori_loop(0, NCHUNKS // 2 - 1, _pair, 0)
    _pair_steps(NCHUNKS - 2, False)
  else:
    lax.fori_loop(0, (NCHUNKS - 1) // 2, _pair, 0)
    _wait_loads(NCHUNKS - 1, 0)
    _start_gather(0)
    _wait_gather(0)
    _compute(0)
    _scatter(0)

  plsc.subcore_barrier()

  row0 = sid * ROWS_PER_SUB
  pltpu.sync_copy(aggr_sh.at[pl.ds(row0, ROWS_PER_SUB)],
                  out_hbm.at[pl.ds(cid * AGG_ROWS + row0, ROWS_PER_SUB)])


def _sc_edge_pass(z, src, dst, ea):
  mesh = plsc.VectorSubcoreMesh(
      core_axis_name="c", subcore_axis_name="s",
      num_cores=NUM_CORES, num_subcores=NUM_SUBCORES)
  call = pl.kernel(
      _sc_edge_body,
      out_type=jax.ShapeDtypeStruct((NUM_CORES * AGG_ROWS, LATENT),
                                    jnp.float32),
      mesh=mesh,
      scratch_types=[
          pltpu.VMEM((CHUNK,), jnp.int32),
          pltpu.VMEM((CHUNK,), jnp.int32),
          pltpu.VMEM((CHUNK, LATENT), jnp.bfloat16),
          pltpu.VMEM((CHUNK, LATENT), jnp.float32),
          pltpu.VMEM((CHUNK,), jnp.int32),
          pltpu.VMEM((CHUNK,), jnp.int32),
          pltpu.VMEM((CHUNK, LATENT), jnp.bfloat16),
          pltpu.VMEM((CHUNK, LATENT), jnp.float32),
          pltpu.VMEM_SHARED((AGG_ROWS, LATENT), jnp.float32),
          pltpu.SemaphoreType.DMA,
          pltpu.SemaphoreType.DMA,
          pltpu.SemaphoreType.DMA,
          pltpu.SemaphoreType.DMA,
      ],
      compiler_params=pltpu.CompilerParams(
          use_tc_tiling_on_sc=False,
          needs_layout_passes=False,
          internal_scratch_in_bytes=64 * 1024),
  )
  return call(z, src, dst, ea)


def kernel(x, y, edge_index, edge_attr, W_node_in, W_edge, gcn_W1, gcn_W2,
           W_node_out):
  src = edge_index[0]
  dst = edge_index[1]
  ea = _edge_embed(edge_attr, W_edge[:, jnp.array(_EA_PERM)])
  z = _matmul(x, W_node_in)
  for i in range(N_AGGR):
    aggr = _sc_edge_pass(z, src, dst, ea)
    z = _mlp(z, aggr, gcn_W1[i], gcn_W2[i])
  y_predict = _matmul(z, W_node_out)
  return (y, y_predict)
